# baseline (speedup 1.0000x reference)
import functools

import jax
import jax.numpy as jnp
from jax import lax
from jax.experimental import pallas as pl
from jax.experimental.pallas import tpu as pltpu
from jax.experimental.pallas import tpu_sc as plsc

B = 16384
F = 26
K = 16
FEAT = 100000
FLEN = F * FEAT
HW = FEAT // 2
PW = 51200
SCW = PW // 16
NGEN = 2 * F
GB_MAX = 2600064 - PW
WB_MAX = FLEN - PW
NC = 2
NS = 16
BH = B // 2
NVEC = BH // 16
BSL = B // NS

_mesh = plsc.VectorSubcoreMesh(core_axis_name="c", subcore_axis_name="s")


@functools.partial(
    pl.kernel,
    out_type=jax.ShapeDtypeStruct((NC, B), jnp.float32),
    mesh=_mesh,
    compiler_params=pltpu.CompilerParams(needs_layout_passes=False),
    scratch_types=[
        pltpu.VMEM_SHARED((8, PW), jnp.float32),
        pltpu.VMEM_SHARED((8, PW), jnp.float32),
        pltpu.VMEM((PW,), jnp.float32),
        pltpu.VMEM((BH,), jnp.int32),
        pltpu.VMEM((BH,), jnp.float32),
        pltpu.VMEM((BH,), jnp.float32),
        pltpu.VMEM((BSL,), jnp.float32),
        pltpu.VMEM((16,), jnp.float32),
        pltpu.HBM((NC * NS, BH), jnp.float32),
        pltpu.HBM((NC * NS, BH), jnp.float32),
        pltpu.SemaphoreType.DMA,
    ],
)
def _fm_sc(inp_hbm, w0_hbm, v_t_hbm, out_hbm,
           pool0, pool1, win, inprow, s_acc, a_acc, outv, w0v,
           s_scr, a_scr, stsem):
    pools = (pool0, pool1)
    c = lax.axis_index("c")
    s = lax.axis_index("s")
    wid = c * NS + s
    p = lax.rem(s, 8)
    bh = s // 8
    bbase = bh * BH
    c8 = pl.multiple_of(c * 8, 8)
    half = jnp.float32(0.5)

    pltpu.sync_copy(w0_hbm, w0v.at[pl.ds(0, 1)])

    @plsc.parallel_loop(0, BH, step=16, unroll=4)
    def _zero(i):
        z = jnp.zeros((16,), jnp.float32)
        s_acc[pl.ds(i, 16)] = z
        a_acc[pl.ds(i, 16)] = z

    def pool_base(g):
        f = g // 2
        hg = lax.rem(g, 2)
        start = f * FEAT + hg * HW
        return jnp.minimum((start // 128) * 128, GB_MAX)

    def stripe_copy(g, buf):
        gb = pl.multiple_of(pool_base(g), 128)
        return pltpu.make_async_copy(
            v_t_hbm.at[pl.ds(c8, 8), pl.ds(gb + s * SCW, SCW)],
            pools[buf].at[:, pl.ds(s * SCW, SCW)],
            stsem,
        )

    def gen_task(g, cur, stage_inp):
        f = g // 2
        hg = lax.rem(g, 2)
        lo = hg * HW

        @pl.when(g + 1 < NGEN)
        def _prefetch():
            stripe_copy(g + 1, 1 - cur).start()

        if stage_inp:
            pltpu.sync_copy(inp_hbm.at[pl.ds(f * B + bbase, BH)], inprow)
        pltpu.sync_copy(pools[cur].at[p], win)

        off = f * FEAT - pool_base(g)

        @plsc.parallel_loop(0, BH, step=16, unroll=4)
        def _body(i):
            idx = inprow[pl.ds(i, 16)]
            mrel = idx - lo
            m = (mrel >= 0) & (mrel < HW)
            v = plsc.load_gather(win, [jnp.where(m, idx + off, 0)])
            v = jnp.where(m, v, jnp.float32(0.0))
            o = pl.ds(i, 16)
            s_acc[o] = s_acc[o] + v
            a_acc[o] = a_acc[o] - half * (v * v)

        @pl.when(g + 1 < NGEN)
        def _drain():
            stripe_copy(g + 1, 1 - cur).wait()

        plsc.subcore_barrier()

    stripe_copy(0, 0).start()
    stripe_copy(0, 0).wait()
    plsc.subcore_barrier()

    def pair(i, carry):
        gen_task(2 * i, 0, stage_inp=True)
        gen_task(2 * i + 1, 1, stage_inp=False)
        return carry

    lax.fori_loop(0, NGEN // 2, pair, 0)

    pltpu.sync_copy(s_acc, s_scr.at[wid])
    pltpu.sync_copy(a_acc, a_scr.at[wid])
    plsc.subcore_barrier()

    bs = s * BSL
    bhm = s // 8
    o8 = lax.rem(s, 8) * BSL
    for q in range(8):
        pltpu.sync_copy(a_scr.at[c * NS + bhm * 8 + q, pl.ds(o8, BSL)],
                        win.at[pl.ds(q * BSL, BSL)])
    for q in range(8):
        pltpu.sync_copy(s_scr.at[c * NS + bhm * 8 + q, pl.ds(o8, BSL)],
                        win.at[pl.ds((8 + q) * BSL, BSL)])

    w0s = w0v[pl.ds(0, 16)][0]
    w0_eff = jnp.where(c == 0, w0s, jnp.float32(0.0))

    def fin(j, carry):
        acc = jnp.full((16,), w0_eff, jnp.float32)
        for q in range(8):
            acc = acc + win[pl.ds(q * BSL + j * 16, 16)]
        for q in range(8):
            sq = win[pl.ds((8 + q) * BSL + j * 16, 16)]
            acc = acc + half * (sq * sq)
        outv[pl.ds(j * 16, 16)] = acc
        return carry

    lax.fori_loop(0, BSL // 16, fin, 0)

    pltpu.sync_copy(outv, out_hbm.at[c, pl.ds(bs, BSL)])


@functools.partial(
    pl.kernel,
    out_type=jax.ShapeDtypeStruct((NC, B), jnp.float32),
    mesh=_mesh,
    compiler_params=pltpu.CompilerParams(needs_layout_passes=False),
    scratch_types=[
        pltpu.VMEM((PW,), jnp.float32),
        pltpu.VMEM((BH,), jnp.int32),
        pltpu.VMEM((BH,), jnp.float32),
        pltpu.VMEM((BSL,), jnp.float32),
        pltpu.HBM((NC * NS, BH), jnp.float32),
    ],
)
def _fm_w(inp_hbm, w_hbm, out_hbm, win, inprow, wacc, outv, w_scr):
    c = lax.axis_index("c")
    s = lax.axis_index("s")
    wid = c * NS + s
    p = lax.rem(s, 8)
    bh = s // 8
    bbase = bh * BH

    @plsc.parallel_loop(0, BH, step=16, unroll=4)
    def _zero(i):
        wacc[pl.ds(i, 16)] = jnp.zeros((16,), jnp.float32)

    for r in range(4):
        tid = c * 8 + p + 16 * r

        @pl.when(tid < NGEN)
        def _w_task():
            f = tid // 2
            hg = lax.rem(tid, 2)
            lo = hg * HW
            wb = jnp.minimum(f * FEAT + lo, WB_MAX)
            woff = f * FEAT - wb
            pltpu.sync_copy(inp_hbm.at[pl.ds(f * B + bbase, BH)], inprow)
            pltpu.sync_copy(w_hbm.at[pl.ds(wb, PW)], win)

            @plsc.parallel_loop(0, BH, step=16, unroll=4)
            def _wbody(i):
                idx = inprow[pl.ds(i, 16)]
                mrel = idx - lo
                m = (mrel >= 0) & (mrel < HW)
                v = plsc.load_gather(win, [jnp.where(m, idx + woff, 0)])
                v = jnp.where(m, v, jnp.float32(0.0))
                o = pl.ds(i, 16)
                wacc[o] = wacc[o] + v

    pltpu.sync_copy(wacc, w_scr.at[wid])
    plsc.subcore_barrier()

    bs = s * BSL
    bhm = s // 8
    o8 = lax.rem(s, 8) * BSL
    for q in range(8):
        pltpu.sync_copy(w_scr.at[c * NS + bhm * 8 + q, pl.ds(o8, BSL)],
                        win.at[pl.ds(q * BSL, BSL)])

    def fin(j, carry):
        acc = jnp.zeros((16,), jnp.float32)
        for q in range(8):
            acc = acc + win[pl.ds(q * BSL + j * 16, 16)]
        outv[pl.ds(j * 16, 16)] = acc
        return carry

    lax.fori_loop(0, BSL // 16, fin, 0)

    pltpu.sync_copy(outv, out_hbm.at[c, pl.ds(bs, BSL)])


def kernel(inputs, w0, w, V):
    inp_flat = inputs.T.reshape(-1)
    out2 = _fm_sc(inp_flat, w0, V.T)
    outw = _fm_w(inp_flat, w.reshape(-1))
    return (out2[0] + out2[1] + outw[0] + outw[1]).reshape(B, 1)


# speedup vs baseline: 1.0374x; 1.0374x over previous
import functools

import jax
import jax.numpy as jnp
from jax import lax
from jax.experimental import pallas as pl
from jax.experimental.pallas import tpu as pltpu
from jax.experimental.pallas import tpu_sc as plsc

B = 16384
F = 26
K = 16
FEAT = 100000
FLEN = F * FEAT
HW = FEAT // 2
PW = 51200
SCW = PW // 16
NGEN = 2 * F
GB_MAX = 2600064 - PW
WB_MAX = FLEN - PW
NC = 2
NS = 16
BH = B // 2
NVEC = BH // 16
BSL = B // NS

_mesh = plsc.VectorSubcoreMesh(core_axis_name="c", subcore_axis_name="s")


@functools.partial(
    pl.kernel,
    out_type=jax.ShapeDtypeStruct((NC, B), jnp.float32),
    mesh=_mesh,
    compiler_params=pltpu.CompilerParams(needs_layout_passes=False),
    scratch_types=[
        pltpu.VMEM_SHARED((8, PW), jnp.float32),
        pltpu.VMEM_SHARED((8, PW), jnp.float32),
        pltpu.VMEM((PW,), jnp.float32),
        pltpu.VMEM((BH,), jnp.int32),
        pltpu.VMEM((BH,), jnp.float32),
        pltpu.VMEM((BH,), jnp.float32),
        pltpu.VMEM((BSL,), jnp.float32),
        pltpu.VMEM((16,), jnp.float32),
        pltpu.HBM((NC * NS, BH), jnp.float32),
        pltpu.HBM((NC * NS, BH), jnp.float32),
        pltpu.SemaphoreType.DMA,
    ],
)
def _fm_sc(inp_hbm, w0_hbm, w_hbm, v_t_hbm, out_hbm,
           pool0, pool1, win, inprow, s_acc, a_acc, outv, w0v,
           s_scr, a_scr, stsem):
    pools = (pool0, pool1)
    c = lax.axis_index("c")
    s = lax.axis_index("s")
    wid = c * NS + s
    p = lax.rem(s, 8)
    bh = s // 8
    bbase = bh * BH
    c8 = pl.multiple_of(c * 8, 8)
    half = jnp.float32(0.5)

    pltpu.sync_copy(w0_hbm, w0v.at[pl.ds(0, 1)])

    @plsc.parallel_loop(0, BH, step=16, unroll=4)
    def _zero(i):
        z = jnp.zeros((16,), jnp.float32)
        s_acc[pl.ds(i, 16)] = z
        a_acc[pl.ds(i, 16)] = z

    def pool_base(g):
        f = g // 2
        hg = lax.rem(g, 2)
        start = f * FEAT + hg * HW
        return jnp.minimum((start // 128) * 128, GB_MAX)

    def stripe_copy(g, buf):
        gb = pl.multiple_of(pool_base(g), 128)
        return pltpu.make_async_copy(
            v_t_hbm.at[pl.ds(c8, 8), pl.ds(gb + s * SCW, SCW)],
            pools[buf].at[:, pl.ds(s * SCW, SCW)],
            stsem,
        )

    def gen_task(g, cur, stage_inp):
        f = g // 2
        hg = lax.rem(g, 2)
        lo = hg * HW

        @pl.when(g + 1 < NGEN)
        def _prefetch():
            stripe_copy(g + 1, 1 - cur).start()

        if stage_inp:
            pltpu.sync_copy(inp_hbm.at[pl.ds(f * B + bbase, BH)], inprow)
        pltpu.sync_copy(pools[cur].at[p], win)

        off = f * FEAT - pool_base(g)

        @plsc.parallel_loop(0, BH, step=16, unroll=8)
        def _body(i):
            idx = inprow[pl.ds(i, 16)]
            mrel = idx - lo
            m = (mrel >= 0) & (mrel < HW)
            v = plsc.load_gather(win, [jnp.where(m, idx + off, 0)])
            v = jnp.where(m, v, jnp.float32(0.0))
            o = pl.ds(i, 16)
            s_acc[o] = s_acc[o] + v
            a_acc[o] = a_acc[o] - half * (v * v)

        @pl.when(g + 1 < NGEN)
        def _drain():
            stripe_copy(g + 1, 1 - cur).wait()

        plsc.subcore_barrier()

    stripe_copy(0, 0).start()
    stripe_copy(0, 0).wait()
    plsc.subcore_barrier()

    def pair(i, carry):
        gen_task(2 * i, 0, stage_inp=True)
        gen_task(2 * i + 1, 1, stage_inp=False)
        return carry

    lax.fori_loop(0, NGEN // 2, pair, 0)

    for r in range(4):
        tid = c * 8 + p + 16 * r

        @pl.when(tid < NGEN)
        def _w_task():
            f = tid // 2
            hg = lax.rem(tid, 2)
            lo = hg * HW
            wb = jnp.minimum(f * FEAT + lo, WB_MAX)
            woff = f * FEAT - wb
            pltpu.sync_copy(inp_hbm.at[pl.ds(f * B + bbase, BH)], inprow)
            pltpu.sync_copy(w_hbm.at[pl.ds(wb, PW)], win)

            @plsc.parallel_loop(0, BH, step=16, unroll=4)
            def _wbody(i):
                idx = inprow[pl.ds(i, 16)]
                mrel = idx - lo
                m = (mrel >= 0) & (mrel < HW)
                v = plsc.load_gather(win, [jnp.where(m, idx + woff, 0)])
                v = jnp.where(m, v, jnp.float32(0.0))
                o = pl.ds(i, 16)
                a_acc[o] = a_acc[o] + v

    pltpu.sync_copy(s_acc, s_scr.at[wid])
    pltpu.sync_copy(a_acc, a_scr.at[wid])
    plsc.subcore_barrier()

    bs = s * BSL
    bhm = s // 8
    o8 = lax.rem(s, 8) * BSL
    for q in range(8):
        pltpu.sync_copy(a_scr.at[c * NS + bhm * 8 + q, pl.ds(o8, BSL)],
                        win.at[pl.ds(q * BSL, BSL)])
    for q in range(8):
        pltpu.sync_copy(s_scr.at[c * NS + bhm * 8 + q, pl.ds(o8, BSL)],
                        win.at[pl.ds((8 + q) * BSL, BSL)])

    w0s = w0v[pl.ds(0, 16)][0]
    w0_eff = jnp.where(c == 0, w0s, jnp.float32(0.0))

    def fin(j, carry):
        acc = jnp.full((16,), w0_eff, jnp.float32)
        for q in range(8):
            acc = acc + win[pl.ds(q * BSL + j * 16, 16)]
        for q in range(8):
            sq = win[pl.ds((8 + q) * BSL + j * 16, 16)]
            acc = acc + half * (sq * sq)
        outv[pl.ds(j * 16, 16)] = acc
        return carry

    lax.fori_loop(0, BSL // 16, fin, 0)

    pltpu.sync_copy(outv, out_hbm.at[c, pl.ds(bs, BSL)])


def kernel(inputs, w0, w, V):
    out2 = _fm_sc(inputs.T.reshape(-1), w0, w.reshape(-1), V.T)
    return (out2[0] + out2[1]).reshape(B, 1)
